# baseline (speedup 1.0000x reference)
import jax
import jax.numpy as jnp
from jax import lax
from jax.experimental import pallas as pl
from jax.experimental.pallas import tpu as pltpu
from jax.experimental.pallas import tpu_sc as plsc

B = 8
NMAX = 512
H = 16
F = 768
N_TOTAL = 2048
NEG = -1000000000.0

NC = 2
NS = 16
NW = NC * NS
ROWS_PER_W = B * NMAX // NW
ZERO_ROW = N_TOTAL


def _tc_idx(ids_ref, idx_ref):
    ids = ids_ref[...]
    r = lax.broadcasted_iota(jnp.int32, (NW, ROWS_PER_W), 0) * ROWS_PER_W \
        + lax.broadcasted_iota(jnp.int32, (NW, ROWS_PER_W), 1)
    barr = r // NMAX
    k = r - barr * NMAX
    cnt_arr = jnp.zeros((NW, ROWS_PER_W), jnp.int32)
    cb_arr = jnp.zeros((NW, ROWS_PER_W), jnp.int32)
    for b in range(B):
        cnt = jnp.sum((ids == b).astype(jnp.int32))
        cb = jnp.sum((ids < b).astype(jnp.int32))
        cnt_arr = jnp.where(barr == b, cnt, cnt_arr)
        cb_arr = jnp.where(barr == b, cb, cb_arr)
    idx_ref[...] = jnp.where(k < cnt_arr, cb_arr + k, ZERO_ROW)


def _sc_dense(x_hbm, idx_hbm, out_hbm, idx_v, rows_v, sem):
    wid = lax.axis_index("s") * NC + lax.axis_index("c")
    base = wid * ROWS_PER_W
    pltpu.sync_copy(idx_hbm.at[pl.ds(base, ROWS_PER_W)], idx_v)
    pltpu.async_copy(x_hbm.at[idx_v], rows_v, sem).wait()
    pltpu.sync_copy(rows_v, out_hbm.at[pl.ds(base, ROWS_PER_W)])


def _tc_mask(ids_ref, mask_hbm, tiles, sem):
    ids = ids_ref[...]
    col = lax.broadcasted_iota(jnp.int32, (NMAX, NMAX), 1)
    for b in range(B):
        cnt = jnp.sum((ids == b).astype(jnp.int32))
        tiles[b] = jnp.where(col >= cnt, NEG, 0.0)
        for h in range(H):
            pltpu.make_async_copy(tiles.at[b], mask_hbm.at[b, h], sem).start()
    for b in range(B):
        for h in range(H):
            pltpu.make_async_copy(tiles.at[b], mask_hbm.at[b, h], sem).wait()


def kernel(x, batch_ids):
    ids2d = batch_ids.astype(jnp.int32).reshape(16, 128)
    x_aug = jnp.concatenate([x, jnp.zeros((8, F), x.dtype)], axis=0)

    idx = pl.pallas_call(
        _tc_idx,
        in_specs=[pl.BlockSpec((16, 128), lambda: (0, 0))],
        out_specs=pl.BlockSpec((NW, ROWS_PER_W), lambda: (0, 0)),
        out_shape=jax.ShapeDtypeStruct((NW, ROWS_PER_W), jnp.int32),
    )(ids2d).reshape(B * NMAX)

    dense_flat = pl.kernel(
        _sc_dense,
        out_type=jax.ShapeDtypeStruct((B * NMAX, F), x.dtype),
        mesh=plsc.VectorSubcoreMesh(core_axis_name="c", subcore_axis_name="s"),
        scratch_types=[
            pltpu.VMEM((ROWS_PER_W,), jnp.int32),
            pltpu.VMEM((ROWS_PER_W, F), jnp.float32),
            pltpu.SemaphoreType.DMA,
        ],
    )(x_aug, idx)
    dense_x = dense_flat.reshape(B, NMAX, F)

    attn_mask = pl.pallas_call(
        _tc_mask,
        in_specs=[pl.BlockSpec((16, 128), lambda: (0, 0))],
        out_specs=pl.BlockSpec(memory_space=pl.ANY),
        out_shape=jax.ShapeDtypeStruct((B, H, NMAX, NMAX), jnp.float32),
        scratch_shapes=[
            pltpu.VMEM((B, NMAX, NMAX), jnp.float32),
            pltpu.SemaphoreType.DMA,
        ],
    )(ids2d)
    return dense_x, attn_mask


# speedup vs baseline: 2.1871x; 2.1871x over previous
import jax
import jax.numpy as jnp
from jax import lax
from jax.experimental import pallas as pl
from jax.experimental.pallas import tpu as pltpu
from jax.experimental.pallas import tpu_sc as plsc

B = 8
NMAX = 512
H = 16
F = 768
N_TOTAL = 2048
NEG = -1000000000.0

NC = 2
NS = 16
NW = NC * NS
ROWS_PER_W = B * NMAX // NW
ZERO_ROW = N_TOTAL


def _tc_idx(ids_ref, idx_ref):
    ids = ids_ref[...]
    r = lax.broadcasted_iota(jnp.int32, (NW, ROWS_PER_W), 0) * ROWS_PER_W \
        + lax.broadcasted_iota(jnp.int32, (NW, ROWS_PER_W), 1)
    barr = r // NMAX
    k = r - barr * NMAX
    cnt_arr = jnp.zeros((NW, ROWS_PER_W), jnp.int32)
    cb_arr = jnp.zeros((NW, ROWS_PER_W), jnp.int32)
    for b in range(B):
        cnt = jnp.sum((ids == b).astype(jnp.int32))
        cb = jnp.sum((ids < b).astype(jnp.int32))
        cnt_arr = jnp.where(barr == b, cnt, cnt_arr)
        cb_arr = jnp.where(barr == b, cb, cb_arr)
    idx_ref[...] = jnp.where(k < cnt_arr, cb_arr + k, ZERO_ROW + k)


def _sc_dense(x_hbm, idx_hbm, out_hbm, idx_v, rows_v, sem):
    wid = lax.axis_index("s") * NC + lax.axis_index("c")
    base = wid * ROWS_PER_W
    pltpu.sync_copy(idx_hbm.at[pl.ds(base, ROWS_PER_W)], idx_v)
    pltpu.async_copy(x_hbm.at[idx_v], rows_v, sem).wait()
    pltpu.sync_copy(rows_v, out_hbm.at[pl.ds(base, ROWS_PER_W)])


def _tc_mask(ids_ref, mask_hbm, tiles, sem):
    ids = ids_ref[...]
    col = lax.broadcasted_iota(jnp.int32, (NMAX, NMAX), 1)
    for b in range(B):
        cnt = jnp.sum((ids == b).astype(jnp.int32))
        tiles[b] = jnp.where(col >= cnt, NEG, 0.0)
        for h in range(H):
            pltpu.make_async_copy(tiles.at[b], mask_hbm.at[b, h], sem).start()
    for b in range(B):
        for h in range(H):
            pltpu.make_async_copy(tiles.at[b], mask_hbm.at[b, h], sem).wait()


def kernel(x, batch_ids):
    ids2d = batch_ids.astype(jnp.int32).reshape(16, 128)
    x_aug = jnp.concatenate([x, jnp.zeros((NMAX, F), x.dtype)], axis=0)

    idx = pl.pallas_call(
        _tc_idx,
        in_specs=[pl.BlockSpec((16, 128), lambda: (0, 0))],
        out_specs=pl.BlockSpec((NW, ROWS_PER_W), lambda: (0, 0)),
        out_shape=jax.ShapeDtypeStruct((NW, ROWS_PER_W), jnp.int32),
    )(ids2d).reshape(B * NMAX)

    dense_flat = pl.kernel(
        _sc_dense,
        out_type=jax.ShapeDtypeStruct((B * NMAX, F), x.dtype),
        mesh=plsc.VectorSubcoreMesh(core_axis_name="c", subcore_axis_name="s"),
        scratch_types=[
            pltpu.VMEM((ROWS_PER_W,), jnp.int32),
            pltpu.VMEM((ROWS_PER_W, F), jnp.float32),
            pltpu.SemaphoreType.DMA,
        ],
    )(x_aug, idx)
    dense_x = dense_flat.reshape(B, NMAX, F)

    attn_mask = pl.pallas_call(
        _tc_mask,
        in_specs=[pl.BlockSpec((16, 128), lambda: (0, 0))],
        out_specs=pl.BlockSpec(memory_space=pl.ANY),
        out_shape=jax.ShapeDtypeStruct((B, H, NMAX, NMAX), jnp.float32),
        scratch_shapes=[
            pltpu.VMEM((B, NMAX, NMAX), jnp.float32),
            pltpu.SemaphoreType.DMA,
        ],
    )(ids2d)
    return dense_x, attn_mask
